# Initial kernel scaffold; baseline (speedup 1.0000x reference)
import jax
import jax.numpy as jnp
from jax.experimental import pallas as pl


def kernel(x, edge_index, W1, b1, W2, b2, W3, b3, gamma, beta, Wc, bc):
    raise NotImplementedError("write your pallas kernel here")


# speedup vs baseline: 11.3840x; 11.3840x over previous
import functools

import jax
import jax.numpy as jnp
from jax import lax
from jax.experimental import pallas as pl
from jax.experimental.pallas import tpu as pltpu
from jax.experimental.pallas import tpu_sc as plsc

_NC = 2
_NS = 16
_K = 128


def _tile_rows(n):
    rpt8 = (n // (8 * _NS)) * 8
    rem = n - rpt8 * _NS
    return rpt8, rem


def _rowcopy(sid, n, src_at, dst_at):
    rpt8, rem = _tile_rows(n)
    r0 = sid * rpt8
    pltpu.sync_copy(src_at(r0, rpt8), dst_at(r0, rpt8))
    if rem:
        @pl.when(sid == 0)
        def _():
            pltpu.sync_copy(src_at(rpt8 * _NS, rem), dst_at(rpt8 * _NS, rem))


def _mk_agg(n, h, e):
    nw = _NC * _NS
    epw = e // nw
    nch = epw // _K
    tail = epw - nch * _K
    mesh = plsc.VectorSubcoreMesh(core_axis_name="c", subcore_axis_name="s")

    @functools.partial(
        pl.kernel,
        out_type=jax.ShapeDtypeStruct((_NC, n, h), jnp.float32),
        mesh=mesh,
        scratch_types=[
            pltpu.VMEM_SHARED((n, h), jnp.float32),
            pltpu.VMEM((_K,), jnp.int32),
            pltpu.VMEM((_K,), jnp.int32),
            pltpu.VMEM((_K, h), jnp.float32),
            pltpu.VMEM((tail,), jnp.int32),
            pltpu.VMEM((tail,), jnp.int32),
            pltpu.VMEM((tail, h), jnp.float32),
            pltpu.SemaphoreType.DMA,
        ],
    )
    def agg_kernel(hs_hbm, src_hbm, dst_hbm, zeros_hbm, out_hbm,
                   acc, src_v, dst_v, rows_v, src_t, dst_t, rows_t, sem):
        cid = lax.axis_index("c")
        sid = lax.axis_index("s")
        wid = cid * _NS + sid
        base = wid * epw

        @pl.when(cid == 0)
        def _():
            _rowcopy(sid, n,
                     lambda r, m: hs_hbm.at[pl.ds(r, m)],
                     lambda r, m: acc.at[pl.ds(r, m)])

        @pl.when(cid != 0)
        def _():
            _rowcopy(sid, n,
                     lambda r, m: zeros_hbm.at[pl.ds(r, m)],
                     lambda r, m: acc.at[pl.ds(r, m)])

        plsc.subcore_barrier()

        def step(g, carry):
            e0 = base + g * _K
            pltpu.sync_copy(src_hbm.at[pl.ds(e0, _K)], src_v)
            pltpu.sync_copy(dst_hbm.at[pl.ds(e0, _K)], dst_v)
            pltpu.async_copy(hs_hbm.at[src_v], rows_v, sem).wait()
            pltpu.sync_copy(rows_v, acc.at[dst_v], add=True)
            return carry

        lax.fori_loop(0, nch, step, 0)
        if tail:
            e0 = base + nch * _K
            pltpu.sync_copy(src_hbm.at[pl.ds(e0, tail)], src_t)
            pltpu.sync_copy(dst_hbm.at[pl.ds(e0, tail)], dst_t)
            pltpu.async_copy(hs_hbm.at[src_t], rows_t, sem).wait()
            pltpu.sync_copy(rows_t, acc.at[dst_t], add=True)
        plsc.subcore_barrier()
        _rowcopy(sid, n,
                 lambda r, m: acc.at[pl.ds(r, m)],
                 lambda r, m: out_hbm.at[cid, pl.ds(r, m)])

    return agg_kernel


_BLK = 1000


def _pre(degs, x, w):
    n, d_in = x.shape
    hdn = w.shape[1]

    def body(degs_ref, x_ref, w_ref, dinv_ref, hs_ref):
        d = degs_ref[...]
        dinv = lax.rsqrt(d[0, :, 0:1] + d[1, :, 0:1])
        dinv_ref[...] = dinv
        hm = jnp.dot(x_ref[...], w_ref[...],
                     preferred_element_type=jnp.float32)
        hs_ref[...] = hm * dinv

    return pl.pallas_call(
        body,
        grid=(n // _BLK,),
        in_specs=[
            pl.BlockSpec((_NC, _BLK, d_in), lambda i: (0, i, 0)),
            pl.BlockSpec((_BLK, d_in), lambda i: (i, 0)),
            pl.BlockSpec((d_in, hdn), lambda i: (0, 0)),
        ],
        out_specs=[
            pl.BlockSpec((_BLK, 1), lambda i: (i, 0)),
            pl.BlockSpec((_BLK, hdn), lambda i: (i, 0)),
        ],
        out_shape=[
            jax.ShapeDtypeStruct((n, 1), jnp.float32),
            jax.ShapeDtypeStruct((n, hdn), jnp.float32),
        ],
    )(degs, x, w)


def _norm_block(accs, dinv, b, g, be):
    t = (accs[0] + accs[1]) * dinv + b
    t = jnp.maximum(t, 0.0)
    mu = jnp.mean(t, axis=-1, keepdims=True)
    tc = t - mu
    var = jnp.mean(tc * tc, axis=-1, keepdims=True)
    return tc * lax.rsqrt(var + 1e-5) * g + be


def _combine(accs, dinv, b, g, be, w):
    _, n, hdn = accs.shape
    hdn2 = w.shape[1]

    def body(accs_ref, dinv_ref, b_ref, g_ref, be_ref, w_ref, out_ref):
        dinv = dinv_ref[...]
        ln = _norm_block(accs_ref[...], dinv, b_ref[...], g_ref[...],
                         be_ref[...])
        out_ref[...] = jnp.dot(ln, w_ref[...],
                               preferred_element_type=jnp.float32) * dinv

    return pl.pallas_call(
        body,
        grid=(n // _BLK,),
        in_specs=[
            pl.BlockSpec((_NC, _BLK, hdn), lambda i: (0, i, 0)),
            pl.BlockSpec((_BLK, 1), lambda i: (i, 0)),
            pl.BlockSpec((1, hdn), lambda i: (0, 0)),
            pl.BlockSpec((1, hdn), lambda i: (0, 0)),
            pl.BlockSpec((1, hdn), lambda i: (0, 0)),
            pl.BlockSpec((hdn, hdn2), lambda i: (0, 0)),
        ],
        out_specs=pl.BlockSpec((_BLK, hdn2), lambda i: (i, 0)),
        out_shape=jax.ShapeDtypeStruct((n, hdn2), jnp.float32),
    )(accs, dinv, b, g, be, w)


def _final(accs, dinv, b, g, be, wc, bc):
    _, n, hdn = accs.shape
    od = wc.shape[1]

    def body(accs_ref, dinv_ref, b_ref, g_ref, be_ref, wc_ref, bc_ref,
             out_ref):
        ln = _norm_block(accs_ref[...], dinv_ref[...], b_ref[...], g_ref[...],
                         be_ref[...])
        out_ref[...] = jnp.dot(ln, wc_ref[...],
                               preferred_element_type=jnp.float32) + bc_ref[...]

    return pl.pallas_call(
        body,
        grid=(n // _BLK,),
        in_specs=[
            pl.BlockSpec((_NC, _BLK, hdn), lambda i: (0, i, 0)),
            pl.BlockSpec((_BLK, 1), lambda i: (i, 0)),
            pl.BlockSpec((1, hdn), lambda i: (0, 0)),
            pl.BlockSpec((1, hdn), lambda i: (0, 0)),
            pl.BlockSpec((1, hdn), lambda i: (0, 0)),
            pl.BlockSpec((hdn, od), lambda i: (0, 0)),
            pl.BlockSpec((1, od), lambda i: (0, 0)),
        ],
        out_specs=pl.BlockSpec((_BLK, od), lambda i: (i, 0)),
        out_shape=jax.ShapeDtypeStruct((n, od), jnp.float32),
    )(accs, dinv, b, g, be, wc, bc)


def kernel(x, edge_index, W1, b1, W2, b2, W3, b3, gamma, beta, Wc, bc):
    n, _ = x.shape
    e = edge_index.shape[1]
    hdn = W1.shape[1]
    src = edge_index[0]
    dst = edge_index[1]
    zeros2 = jnp.zeros((n, hdn), jnp.float32)
    ones2 = jnp.ones((n, hdn), jnp.float32)
    b1r, b2r, b3r = (v.reshape(1, -1) for v in (b1, b2, b3))
    gr, ber, bcr = gamma.reshape(1, -1), beta.reshape(1, -1), bc.reshape(1, -1)

    agg = _mk_agg(n, hdn, e)
    degs = agg(ones2, src, dst, zeros2)
    dinv, hs = _pre(degs, x, W1)
    accs = agg(hs, src, dst, zeros2)
    hs = _combine(accs, dinv, b1r, gr, ber, W2)
    accs = agg(hs, src, dst, zeros2)
    hs = _combine(accs, dinv, b2r, gr, ber, W3)
    accs = agg(hs, src, dst, zeros2)
    return _final(accs, dinv, b3r, gr, ber, Wc, bcr)


# speedup vs baseline: 22.0028x; 1.9328x over previous
import functools

import jax
import jax.numpy as jnp
from jax import lax
from jax.experimental import pallas as pl
from jax.experimental.pallas import tpu as pltpu
from jax.experimental.pallas import tpu_sc as plsc

_NC = 2
_NS = 16
_K = 128


def _tile_rows(n):
    rpt8 = (n // (8 * _NS)) * 8
    rem = n - rpt8 * _NS
    return rpt8, rem


def _rowcopy(sid, n, src_at, dst_at):
    rpt8, rem = _tile_rows(n)
    r0 = sid * rpt8
    pltpu.sync_copy(src_at(r0, rpt8), dst_at(r0, rpt8))
    if rem:
        @pl.when(sid == 0)
        def _():
            pltpu.sync_copy(src_at(rpt8 * _NS, rem), dst_at(rpt8 * _NS, rem))


def _stage_indices(dst_hbm, base, nch, dsta, semi, extra=()):
    for s, d in extra:
        pltpu.async_copy(s, d, semi)

    def fire(g, c):
        pltpu.async_copy(dst_hbm.at[pl.ds(base + g * _K, _K)],
                         dsta.at[g], semi)
        return c

    lax.fori_loop(0, nch, fire, 0)

    def drain(g, c):
        pltpu.make_async_copy(dst_hbm.at[pl.ds(base + g * _K, _K)],
                              dsta.at[g], semi).wait()
        return c

    for s, d in extra:
        pltpu.make_async_copy(s, d, semi).wait()
    lax.fori_loop(0, nch, drain, 0)


def _mk_agg(n, h, e):
    nw = _NC * _NS
    epw = e // nw
    nch = epw // _K
    tail = epw - nch * _K
    assert nch % 2 == 0
    mesh = plsc.VectorSubcoreMesh(core_axis_name="c", subcore_axis_name="s")

    @functools.partial(
        pl.kernel,
        out_type=jax.ShapeDtypeStruct((_NC, n, h), jnp.float32),
        mesh=mesh,
        scratch_types=[
            pltpu.VMEM_SHARED((n, h), jnp.float32),
            pltpu.VMEM((nch, _K), jnp.int32),
            pltpu.VMEM((_K, h), jnp.float32),
            pltpu.VMEM((_K, h), jnp.float32),
            pltpu.VMEM((_K,), jnp.int32),
            pltpu.VMEM((_K,), jnp.int32),
            pltpu.VMEM((tail,), jnp.int32),
            pltpu.VMEM((tail,), jnp.int32),
            pltpu.VMEM((tail, h), jnp.float32),
            pltpu.SemaphoreType.DMA,
            pltpu.SemaphoreType.DMA,
            pltpu.SemaphoreType.DMA,
            pltpu.SemaphoreType.DMA,
        ],
    )
    def agg_kernel(hs_hbm, src_hbm, dst_hbm, zeros_hbm, out_hbm,
                   acc, dsta, bufa, bufb, sia, sib, src_t, dst_t, rows_t,
                   sema, semb, semia, semib):
        cid = lax.axis_index("c")
        sid = lax.axis_index("s")
        wid = cid * _NS + sid
        base = wid * epw

        @pl.when(cid == 0)
        def _():
            _rowcopy(sid, n,
                     lambda r, m: hs_hbm.at[pl.ds(r, m)],
                     lambda r, m: acc.at[pl.ds(r, m)])

        @pl.when(cid != 0)
        def _():
            _rowcopy(sid, n,
                     lambda r, m: zeros_hbm.at[pl.ds(r, m)],
                     lambda r, m: acc.at[pl.ds(r, m)])

        _stage_indices(dst_hbm, base, nch, dsta, semia)
        plsc.subcore_barrier()

        def stage_src(g, ibuf, sem):
            pltpu.async_copy(src_hbm.at[pl.ds(base + g * _K, _K)], ibuf, sem)

        def wait_src(g, ibuf, sem):
            pltpu.make_async_copy(src_hbm.at[pl.ds(base + g * _K, _K)],
                                  ibuf, sem).wait()

        def gather(buf, ibuf, sem):
            pltpu.async_copy(hs_hbm.at[ibuf], buf, sem)

        def wait_gather(buf, ibuf, sem):
            pltpu.make_async_copy(hs_hbm.at[ibuf], buf, sem).wait()

        def scat(g, buf):
            pltpu.sync_copy(buf, acc.at[dsta.at[g]], add=True)

        stage_src(0, sia, semia)
        stage_src(1, sib, semib)
        wait_src(0, sia, semia)
        gather(bufa, sia, sema)

        def pipe(it, carry):
            g = 2 * it
            wait_gather(bufa, sia, sema)
            wait_src(g + 1, sib, semib)
            gather(bufb, sib, semb)

            @pl.when(g + 2 < nch)
            def _():
                stage_src(g + 2, sia, semia)

            scat(g, bufa)
            wait_gather(bufb, sib, semb)

            @pl.when(g + 2 < nch)
            def _():
                wait_src(g + 2, sia, semia)
                gather(bufa, sia, sema)

            @pl.when(g + 3 < nch)
            def _():
                stage_src(g + 3, sib, semib)

            scat(g + 1, bufb)
            return carry

        lax.fori_loop(0, nch // 2, pipe, 0)
        if tail:
            e0 = base + nch * _K
            pltpu.sync_copy(src_hbm.at[pl.ds(e0, tail)], src_t)
            pltpu.sync_copy(dst_hbm.at[pl.ds(e0, tail)], dst_t)
            pltpu.async_copy(hs_hbm.at[src_t], rows_t, sema).wait()
            pltpu.sync_copy(rows_t, acc.at[dst_t], add=True)
        plsc.subcore_barrier()
        _rowcopy(sid, n,
                 lambda r, m: acc.at[pl.ds(r, m)],
                 lambda r, m: out_hbm.at[cid, pl.ds(r, m)])

    return agg_kernel


def _mk_deg(n, h, e):
    nw = _NC * _NS
    epw = e // nw
    nch = epw // _K
    tail = epw - nch * _K
    mesh = plsc.VectorSubcoreMesh(core_axis_name="c", subcore_axis_name="s")

    @functools.partial(
        pl.kernel,
        out_type=jax.ShapeDtypeStruct((_NC, n, h), jnp.float32),
        mesh=mesh,
        scratch_types=[
            pltpu.VMEM_SHARED((n, h), jnp.float32),
            pltpu.VMEM((nch, _K), jnp.int32),
            pltpu.VMEM((_K, h), jnp.float32),
            pltpu.VMEM((tail,), jnp.int32),
            pltpu.SemaphoreType.DMA,
            pltpu.SemaphoreType.DMA,
        ],
    )
    def deg_kernel(dst_hbm, ones_hbm, zeros_hbm, out_hbm,
                   acc, dsta, ones_v, dst_t, sems, semi):
        cid = lax.axis_index("c")
        sid = lax.axis_index("s")
        wid = cid * _NS + sid
        base = wid * epw

        _rowcopy(sid, n,
                 lambda r, m: zeros_hbm.at[pl.ds(r, m)],
                 lambda r, m: acc.at[pl.ds(r, m)])
        _stage_indices(dst_hbm, base, nch, dsta, semi,
                       extra=[(ones_hbm, ones_v)])
        plsc.subcore_barrier()

        def fire(g, c):
            pltpu.async_copy(ones_v, acc.at[dsta.at[g]], sems, add=True)
            return c

        lax.fori_loop(0, nch, fire, 0)

        def drain(g, c):
            pltpu.make_async_copy(ones_v, acc.at[dsta.at[g]], sems).wait()
            return c

        lax.fori_loop(0, nch, drain, 0)
        if tail:
            pltpu.sync_copy(dst_hbm.at[pl.ds(base + nch * _K, tail)], dst_t)
            pltpu.sync_copy(ones_v.at[pl.ds(0, tail)], acc.at[dst_t],
                            add=True)
        plsc.subcore_barrier()
        _rowcopy(sid, n,
                 lambda r, m: acc.at[pl.ds(r, m)],
                 lambda r, m: out_hbm.at[cid, pl.ds(r, m)])

    return deg_kernel


_BLK = 1000


def _pre(degs, x, w):
    n, d_in = x.shape
    hdn = w.shape[1]

    def body(degs_ref, x_ref, w_ref, dinv_ref, hs_ref):
        d = degs_ref[...]
        dinv = lax.rsqrt(d[0, :, 0:1] + d[1, :, 0:1] + 1.0)
        dinv_ref[...] = dinv
        hm = jnp.dot(x_ref[...], w_ref[...],
                     preferred_element_type=jnp.float32)
        hs_ref[...] = hm * dinv

    return pl.pallas_call(
        body,
        grid=(n // _BLK,),
        in_specs=[
            pl.BlockSpec((_NC, _BLK, d_in), lambda i: (0, i, 0)),
            pl.BlockSpec((_BLK, d_in), lambda i: (i, 0)),
            pl.BlockSpec((d_in, hdn), lambda i: (0, 0)),
        ],
        out_specs=[
            pl.BlockSpec((_BLK, 1), lambda i: (i, 0)),
            pl.BlockSpec((_BLK, hdn), lambda i: (i, 0)),
        ],
        out_shape=[
            jax.ShapeDtypeStruct((n, 1), jnp.float32),
            jax.ShapeDtypeStruct((n, hdn), jnp.float32),
        ],
    )(degs, x, w)


def _norm_block(accs, dinv, b, g, be):
    t = (accs[0] + accs[1]) * dinv + b
    t = jnp.maximum(t, 0.0)
    mu = jnp.mean(t, axis=-1, keepdims=True)
    tc = t - mu
    var = jnp.mean(tc * tc, axis=-1, keepdims=True)
    return tc * lax.rsqrt(var + 1e-5) * g + be


def _combine(accs, dinv, b, g, be, w):
    _, n, hdn = accs.shape
    hdn2 = w.shape[1]

    def body(accs_ref, dinv_ref, b_ref, g_ref, be_ref, w_ref, out_ref):
        dinv = dinv_ref[...]
        ln = _norm_block(accs_ref[...], dinv, b_ref[...], g_ref[...],
                         be_ref[...])
        out_ref[...] = jnp.dot(ln, w_ref[...],
                               preferred_element_type=jnp.float32) * dinv

    return pl.pallas_call(
        body,
        grid=(n // _BLK,),
        in_specs=[
            pl.BlockSpec((_NC, _BLK, hdn), lambda i: (0, i, 0)),
            pl.BlockSpec((_BLK, 1), lambda i: (i, 0)),
            pl.BlockSpec((1, hdn), lambda i: (0, 0)),
            pl.BlockSpec((1, hdn), lambda i: (0, 0)),
            pl.BlockSpec((1, hdn), lambda i: (0, 0)),
            pl.BlockSpec((hdn, hdn2), lambda i: (0, 0)),
        ],
        out_specs=pl.BlockSpec((_BLK, hdn2), lambda i: (i, 0)),
        out_shape=jax.ShapeDtypeStruct((n, hdn2), jnp.float32),
    )(accs, dinv, b, g, be, w)


def _final(accs, dinv, b, g, be, wc, bc):
    _, n, hdn = accs.shape
    od = wc.shape[1]

    def body(accs_ref, dinv_ref, b_ref, g_ref, be_ref, wc_ref, bc_ref,
             out_ref):
        ln = _norm_block(accs_ref[...], dinv_ref[...], b_ref[...], g_ref[...],
                         be_ref[...])
        out_ref[...] = jnp.dot(ln, wc_ref[...],
                               preferred_element_type=jnp.float32) + bc_ref[...]

    return pl.pallas_call(
        body,
        grid=(n // _BLK,),
        in_specs=[
            pl.BlockSpec((_NC, _BLK, hdn), lambda i: (0, i, 0)),
            pl.BlockSpec((_BLK, 1), lambda i: (i, 0)),
            pl.BlockSpec((1, hdn), lambda i: (0, 0)),
            pl.BlockSpec((1, hdn), lambda i: (0, 0)),
            pl.BlockSpec((1, hdn), lambda i: (0, 0)),
            pl.BlockSpec((hdn, od), lambda i: (0, 0)),
            pl.BlockSpec((1, od), lambda i: (0, 0)),
        ],
        out_specs=pl.BlockSpec((_BLK, od), lambda i: (i, 0)),
        out_shape=jax.ShapeDtypeStruct((n, od), jnp.float32),
    )(accs, dinv, b, g, be, wc, bc)


def kernel(x, edge_index, W1, b1, W2, b2, W3, b3, gamma, beta, Wc, bc):
    n, _ = x.shape
    e = edge_index.shape[1]
    hdn = W1.shape[1]
    src = edge_index[0]
    dst = edge_index[1]
    zeros2 = jnp.zeros((n, hdn), jnp.float32)
    onesk = jnp.ones((_K, hdn), jnp.float32)
    b1r, b2r, b3r = (v.reshape(1, -1) for v in (b1, b2, b3))
    gr, ber, bcr = gamma.reshape(1, -1), beta.reshape(1, -1), bc.reshape(1, -1)

    agg = _mk_agg(n, hdn, e)
    degs = _mk_deg(n, hdn, e)(dst, onesk, zeros2)
    dinv, hs = _pre(degs, x, W1)
    accs = agg(hs, src, dst, zeros2)
    hs = _combine(accs, dinv, b1r, gr, ber, W2)
    accs = agg(hs, src, dst, zeros2)
    hs = _combine(accs, dinv, b2r, gr, ber, W3)
    accs = agg(hs, src, dst, zeros2)
    return _final(accs, dinv, b3r, gr, ber, Wc, bcr)


# speedup vs baseline: 22.0700x; 1.0031x over previous
import functools

import jax
import jax.numpy as jnp
from jax import lax
from jax.experimental import pallas as pl
from jax.experimental.pallas import tpu as pltpu
from jax.experimental.pallas import tpu_sc as plsc

_NC = 2
_NS = 16
_K = 128


def _tile_rows(n):
    rpt8 = (n // (8 * _NS)) * 8
    rem = n - rpt8 * _NS
    return rpt8, rem


def _rowcopy(sid, n, src_at, dst_at):
    rpt8, rem = _tile_rows(n)
    r0 = sid * rpt8
    pltpu.sync_copy(src_at(r0, rpt8), dst_at(r0, rpt8))
    if rem:
        @pl.when(sid == 0)
        def _():
            pltpu.sync_copy(src_at(rpt8 * _NS, rem), dst_at(rpt8 * _NS, rem))


def _stage_indices(dst_hbm, base, nch, dsta, semi, extra=()):
    for s, d in extra:
        pltpu.async_copy(s, d, semi)

    def fire(g, c):
        pltpu.async_copy(dst_hbm.at[pl.ds(base + g * _K, _K)],
                         dsta.at[g], semi)
        return c

    lax.fori_loop(0, nch, fire, 0)

    def drain(g, c):
        pltpu.make_async_copy(dst_hbm.at[pl.ds(base + g * _K, _K)],
                              dsta.at[g], semi).wait()
        return c

    for s, d in extra:
        pltpu.make_async_copy(s, d, semi).wait()
    lax.fori_loop(0, nch, drain, 0)


def _mk_agg(n, h, e):
    nw = _NC * _NS
    epw = e // nw
    nch = epw // _K
    tail = epw - nch * _K
    assert nch % 2 == 0
    mesh = plsc.VectorSubcoreMesh(core_axis_name="c", subcore_axis_name="s")

    @functools.partial(
        pl.kernel,
        out_type=jax.ShapeDtypeStruct((_NC, n, h), jnp.float32),
        mesh=mesh,
        scratch_types=[
            pltpu.VMEM_SHARED((n, h), jnp.float32),
            pltpu.VMEM((nch, _K), jnp.int32),
            pltpu.VMEM((_K, h), jnp.float32),
            pltpu.VMEM((_K, h), jnp.float32),
            pltpu.VMEM((_K,), jnp.int32),
            pltpu.VMEM((_K,), jnp.int32),
            pltpu.VMEM((tail,), jnp.int32),
            pltpu.VMEM((tail,), jnp.int32),
            pltpu.VMEM((tail, h), jnp.float32),
            pltpu.SemaphoreType.DMA,
            pltpu.SemaphoreType.DMA,
            pltpu.SemaphoreType.DMA,
            pltpu.SemaphoreType.DMA,
            pltpu.SemaphoreType.DMA,
            pltpu.SemaphoreType.DMA,
        ],
    )
    def agg_kernel(hs_hbm, src_hbm, dst_hbm, zeros_hbm, out_hbm,
                   acc, dsta, bufa, bufb, sia, sib, src_t, dst_t, rows_t,
                   sema, semb, semia, semib, semsa, semsb):
        cid = lax.axis_index("c")
        sid = lax.axis_index("s")
        wid = cid * _NS + sid
        base = wid * epw

        @pl.when(cid == 0)
        def _():
            _rowcopy(sid, n,
                     lambda r, m: hs_hbm.at[pl.ds(r, m)],
                     lambda r, m: acc.at[pl.ds(r, m)])

        @pl.when(cid != 0)
        def _():
            _rowcopy(sid, n,
                     lambda r, m: zeros_hbm.at[pl.ds(r, m)],
                     lambda r, m: acc.at[pl.ds(r, m)])

        _stage_indices(dst_hbm, base, nch, dsta, semia)
        plsc.subcore_barrier()

        def stage_src(g, ibuf, sem):
            pltpu.async_copy(src_hbm.at[pl.ds(base + g * _K, _K)], ibuf, sem)

        def wait_src(g, ibuf, sem):
            pltpu.make_async_copy(src_hbm.at[pl.ds(base + g * _K, _K)],
                                  ibuf, sem).wait()

        def gather(buf, ibuf, sem):
            pltpu.async_copy(hs_hbm.at[ibuf], buf, sem)

        def wait_gather(buf, ibuf, sem):
            pltpu.make_async_copy(hs_hbm.at[ibuf], buf, sem).wait()

        def scat(g, buf, sem):
            pltpu.async_copy(buf, acc.at[dsta.at[g]], sem, add=True)

        def wait_scat(g, buf, sem):
            pltpu.make_async_copy(buf, acc.at[dsta.at[g]], sem).wait()

        stage_src(0, sia, semia)
        stage_src(1, sib, semib)
        wait_src(0, sia, semia)
        gather(bufa, sia, sema)

        def pipe(it, carry):
            g = 2 * it
            wait_gather(bufa, sia, sema)
            scat(g, bufa, semsa)

            @pl.when(it > 0)
            def _():
                wait_scat(g - 1, bufb, semsb)

            wait_src(g + 1, sib, semib)
            gather(bufb, sib, semb)

            @pl.when(g + 2 < nch)
            def _():
                stage_src(g + 2, sia, semia)

            wait_gather(bufb, sib, semb)
            scat(g + 1, bufb, semsb)
            wait_scat(g, bufa, semsa)

            @pl.when(g + 2 < nch)
            def _():
                wait_src(g + 2, sia, semia)
                gather(bufa, sia, sema)

            @pl.when(g + 3 < nch)
            def _():
                stage_src(g + 3, sib, semib)

            return carry

        lax.fori_loop(0, nch // 2, pipe, 0)
        wait_scat(nch - 1, bufb, semsb)
        if tail:
            e0 = base + nch * _K
            pltpu.sync_copy(src_hbm.at[pl.ds(e0, tail)], src_t)
            pltpu.sync_copy(dst_hbm.at[pl.ds(e0, tail)], dst_t)
            pltpu.async_copy(hs_hbm.at[src_t], rows_t, sema).wait()
            pltpu.sync_copy(rows_t, acc.at[dst_t], add=True)
        plsc.subcore_barrier()
        _rowcopy(sid, n,
                 lambda r, m: acc.at[pl.ds(r, m)],
                 lambda r, m: out_hbm.at[cid, pl.ds(r, m)])

    return agg_kernel


def _mk_deg(n, h, e):
    nw = _NC * _NS
    epw = e // nw
    nch = epw // _K
    tail = epw - nch * _K
    mesh = plsc.VectorSubcoreMesh(core_axis_name="c", subcore_axis_name="s")

    @functools.partial(
        pl.kernel,
        out_type=jax.ShapeDtypeStruct((_NC, n, h), jnp.float32),
        mesh=mesh,
        scratch_types=[
            pltpu.VMEM_SHARED((n, h), jnp.float32),
            pltpu.VMEM((nch, _K), jnp.int32),
            pltpu.VMEM((_K, h), jnp.float32),
            pltpu.VMEM((tail,), jnp.int32),
            pltpu.SemaphoreType.DMA,
            pltpu.SemaphoreType.DMA,
        ],
    )
    def deg_kernel(dst_hbm, ones_hbm, zeros_hbm, out_hbm,
                   acc, dsta, ones_v, dst_t, sems, semi):
        cid = lax.axis_index("c")
        sid = lax.axis_index("s")
        wid = cid * _NS + sid
        base = wid * epw

        _rowcopy(sid, n,
                 lambda r, m: zeros_hbm.at[pl.ds(r, m)],
                 lambda r, m: acc.at[pl.ds(r, m)])
        _stage_indices(dst_hbm, base, nch, dsta, semi,
                       extra=[(ones_hbm, ones_v)])
        plsc.subcore_barrier()

        def fire(g, c):
            pltpu.async_copy(ones_v, acc.at[dsta.at[g]], sems, add=True)
            return c

        lax.fori_loop(0, nch, fire, 0)

        def drain(g, c):
            pltpu.make_async_copy(ones_v, acc.at[dsta.at[g]], sems).wait()
            return c

        lax.fori_loop(0, nch, drain, 0)
        if tail:
            pltpu.sync_copy(dst_hbm.at[pl.ds(base + nch * _K, tail)], dst_t)
            pltpu.sync_copy(ones_v.at[pl.ds(0, tail)], acc.at[dst_t],
                            add=True)
        plsc.subcore_barrier()
        _rowcopy(sid, n,
                 lambda r, m: acc.at[pl.ds(r, m)],
                 lambda r, m: out_hbm.at[cid, pl.ds(r, m)])

    return deg_kernel


_BLK = 1000


def _pre(degs, x, w):
    n, d_in = x.shape
    hdn = w.shape[1]

    def body(degs_ref, x_ref, w_ref, dinv_ref, hs_ref):
        d = degs_ref[...]
        dinv = lax.rsqrt(d[0, :, 0:1] + d[1, :, 0:1] + 1.0)
        dinv_ref[...] = dinv
        hm = jnp.dot(x_ref[...], w_ref[...],
                     preferred_element_type=jnp.float32)
        hs_ref[...] = hm * dinv

    return pl.pallas_call(
        body,
        grid=(n // _BLK,),
        in_specs=[
            pl.BlockSpec((_NC, _BLK, d_in), lambda i: (0, i, 0)),
            pl.BlockSpec((_BLK, d_in), lambda i: (i, 0)),
            pl.BlockSpec((d_in, hdn), lambda i: (0, 0)),
        ],
        out_specs=[
            pl.BlockSpec((_BLK, 1), lambda i: (i, 0)),
            pl.BlockSpec((_BLK, hdn), lambda i: (i, 0)),
        ],
        out_shape=[
            jax.ShapeDtypeStruct((n, 1), jnp.float32),
            jax.ShapeDtypeStruct((n, hdn), jnp.float32),
        ],
    )(degs, x, w)


def _norm_block(accs, dinv, b, g, be):
    t = (accs[0] + accs[1]) * dinv + b
    t = jnp.maximum(t, 0.0)
    mu = jnp.mean(t, axis=-1, keepdims=True)
    tc = t - mu
    var = jnp.mean(tc * tc, axis=-1, keepdims=True)
    return tc * lax.rsqrt(var + 1e-5) * g + be


def _combine(accs, dinv, b, g, be, w):
    _, n, hdn = accs.shape
    hdn2 = w.shape[1]

    def body(accs_ref, dinv_ref, b_ref, g_ref, be_ref, w_ref, out_ref):
        dinv = dinv_ref[...]
        ln = _norm_block(accs_ref[...], dinv, b_ref[...], g_ref[...],
                         be_ref[...])
        out_ref[...] = jnp.dot(ln, w_ref[...],
                               preferred_element_type=jnp.float32) * dinv

    return pl.pallas_call(
        body,
        grid=(n // _BLK,),
        in_specs=[
            pl.BlockSpec((_NC, _BLK, hdn), lambda i: (0, i, 0)),
            pl.BlockSpec((_BLK, 1), lambda i: (i, 0)),
            pl.BlockSpec((1, hdn), lambda i: (0, 0)),
            pl.BlockSpec((1, hdn), lambda i: (0, 0)),
            pl.BlockSpec((1, hdn), lambda i: (0, 0)),
            pl.BlockSpec((hdn, hdn2), lambda i: (0, 0)),
        ],
        out_specs=pl.BlockSpec((_BLK, hdn2), lambda i: (i, 0)),
        out_shape=jax.ShapeDtypeStruct((n, hdn2), jnp.float32),
    )(accs, dinv, b, g, be, w)


def _final(accs, dinv, b, g, be, wc, bc):
    _, n, hdn = accs.shape
    od = wc.shape[1]

    def body(accs_ref, dinv_ref, b_ref, g_ref, be_ref, wc_ref, bc_ref,
             out_ref):
        ln = _norm_block(accs_ref[...], dinv_ref[...], b_ref[...], g_ref[...],
                         be_ref[...])
        out_ref[...] = jnp.dot(ln, wc_ref[...],
                               preferred_element_type=jnp.float32) + bc_ref[...]

    return pl.pallas_call(
        body,
        grid=(n // _BLK,),
        in_specs=[
            pl.BlockSpec((_NC, _BLK, hdn), lambda i: (0, i, 0)),
            pl.BlockSpec((_BLK, 1), lambda i: (i, 0)),
            pl.BlockSpec((1, hdn), lambda i: (0, 0)),
            pl.BlockSpec((1, hdn), lambda i: (0, 0)),
            pl.BlockSpec((1, hdn), lambda i: (0, 0)),
            pl.BlockSpec((hdn, od), lambda i: (0, 0)),
            pl.BlockSpec((1, od), lambda i: (0, 0)),
        ],
        out_specs=pl.BlockSpec((_BLK, od), lambda i: (i, 0)),
        out_shape=jax.ShapeDtypeStruct((n, od), jnp.float32),
    )(accs, dinv, b, g, be, wc, bc)


def kernel(x, edge_index, W1, b1, W2, b2, W3, b3, gamma, beta, Wc, bc):
    n, _ = x.shape
    e = edge_index.shape[1]
    hdn = W1.shape[1]
    src = edge_index[0]
    dst = edge_index[1]
    zeros2 = jnp.zeros((n, hdn), jnp.float32)
    onesk = jnp.ones((_K, hdn), jnp.float32)
    b1r, b2r, b3r = (v.reshape(1, -1) for v in (b1, b2, b3))
    gr, ber, bcr = gamma.reshape(1, -1), beta.reshape(1, -1), bc.reshape(1, -1)

    agg = _mk_agg(n, hdn, e)
    degs = _mk_deg(n, hdn, e)(dst, onesk, zeros2)
    dinv, hs = _pre(degs, x, W1)
    accs = agg(hs, src, dst, zeros2)
    hs = _combine(accs, dinv, b1r, gr, ber, W2)
    accs = agg(hs, src, dst, zeros2)
    hs = _combine(accs, dinv, b2r, gr, ber, W3)
    accs = agg(hs, src, dst, zeros2)
    return _final(accs, dinv, b3r, gr, ber, Wc, bcr)


# speedup vs baseline: 22.2102x; 1.0064x over previous
import functools

import jax
import jax.numpy as jnp
from jax import lax
from jax.experimental import pallas as pl
from jax.experimental.pallas import tpu as pltpu
from jax.experimental.pallas import tpu_sc as plsc

_NC = 2
_NS = 16
_K = 128


def _tile_rows(n):
    rpt8 = (n // (8 * _NS)) * 8
    rem = n - rpt8 * _NS
    return rpt8, rem


def _rowcopy(sid, n, src_at, dst_at):
    rpt8, rem = _tile_rows(n)
    r0 = sid * rpt8
    pltpu.sync_copy(src_at(r0, rpt8), dst_at(r0, rpt8))
    if rem:
        @pl.when(sid == 0)
        def _():
            pltpu.sync_copy(src_at(rpt8 * _NS, rem), dst_at(rpt8 * _NS, rem))


def _stage_indices(dst_hbm, base, nch, dsta, semi, extra=()):
    for s, d in extra:
        pltpu.async_copy(s, d, semi)

    def fire(g, c):
        pltpu.async_copy(dst_hbm.at[pl.ds(base + g * _K, _K)],
                         dsta.at[g], semi)
        return c

    lax.fori_loop(0, nch, fire, 0)

    def drain(g, c):
        pltpu.make_async_copy(dst_hbm.at[pl.ds(base + g * _K, _K)],
                              dsta.at[g], semi).wait()
        return c

    for s, d in extra:
        pltpu.make_async_copy(s, d, semi).wait()
    lax.fori_loop(0, nch, drain, 0)


def _mk_agg(n, h, e):
    nw = _NC * _NS
    epw = e // nw
    nch = epw // _K
    tail = epw - nch * _K
    assert nch % 2 == 0
    mesh = plsc.VectorSubcoreMesh(core_axis_name="c", subcore_axis_name="s")

    assert nch % 3 == 0
    mesh = plsc.VectorSubcoreMesh(core_axis_name="c", subcore_axis_name="s")

    @functools.partial(
        pl.kernel,
        out_type=jax.ShapeDtypeStruct((_NC, n, h), jnp.float32),
        mesh=mesh,
        scratch_types=[
            pltpu.VMEM_SHARED((n, h), jnp.float32),
            [pltpu.VMEM((_K, h), jnp.float32) for _ in range(3)],
            [pltpu.VMEM((_K,), jnp.int32) for _ in range(3)],
            [pltpu.VMEM((_K,), jnp.int32) for _ in range(3)],
            pltpu.VMEM((tail,), jnp.int32),
            pltpu.VMEM((tail,), jnp.int32),
            [pltpu.SemaphoreType.DMA for _ in range(3)],
            [pltpu.SemaphoreType.DMA for _ in range(3)],
            [pltpu.SemaphoreType.DMA for _ in range(3)],
        ],
    )
    def agg_kernel(hs_hbm, src_hbm, dst_hbm, zeros_hbm, out_hbm,
                   acc, bufs, sis, dis, src_t, dst_t, semg, semi, sems):
        cid = lax.axis_index("c")
        sid = lax.axis_index("s")
        wid = cid * _NS + sid
        base = wid * epw

        @pl.when(cid == 0)
        def _():
            _rowcopy(sid, n,
                     lambda r, m: hs_hbm.at[pl.ds(r, m)],
                     lambda r, m: acc.at[pl.ds(r, m)])

        @pl.when(cid != 0)
        def _():
            _rowcopy(sid, n,
                     lambda r, m: zeros_hbm.at[pl.ds(r, m)],
                     lambda r, m: acc.at[pl.ds(r, m)])

        plsc.subcore_barrier()

        def stage(g, x):
            pltpu.async_copy(src_hbm.at[pl.ds(base + g * _K, _K)],
                             sis[x], semi[x])
            pltpu.async_copy(dst_hbm.at[pl.ds(base + g * _K, _K)],
                             dis[x], semi[x])

        def gather(g, x):
            pltpu.make_async_copy(src_hbm.at[pl.ds(base + g * _K, _K)],
                                  sis[x], semi[x]).wait()
            pltpu.make_async_copy(dst_hbm.at[pl.ds(base + g * _K, _K)],
                                  dis[x], semi[x]).wait()
            pltpu.async_copy(hs_hbm.at[sis[x]], bufs[x], semg[x])

        def scat(g, x):
            pltpu.make_async_copy(hs_hbm.at[sis[x]], bufs[x], semg[x]).wait()
            pltpu.async_copy(bufs[x], acc.at[dis[x]], sems[x], add=True)

        def drain(g, x):
            pltpu.make_async_copy(bufs[x], acc.at[dis[x]], sems[x]).wait()

        stage(0, 0)
        stage(1, 1)
        gather(0, 0)

        def pipe(it, carry):
            g = 3 * it
            scat(g, 0)
            gather(g + 1, 1)

            @pl.when(it > 0)
            def _():
                drain(g - 1, 2)

            stage(g + 2, 2)
            scat(g + 1, 1)
            gather(g + 2, 2)
            drain(g, 0)

            @pl.when(g + 3 < nch)
            def _():
                stage(g + 3, 0)

            scat(g + 2, 2)

            @pl.when(g + 3 < nch)
            def _():
                gather(g + 3, 0)

            drain(g + 1, 1)

            @pl.when(g + 4 < nch)
            def _():
                stage(g + 4, 1)

            return carry

        lax.fori_loop(0, nch // 3, pipe, 0)
        drain(nch - 1, 2)
        if tail:
            e0 = base + nch * _K
            pltpu.sync_copy(src_hbm.at[pl.ds(e0, tail)], src_t)
            pltpu.sync_copy(dst_hbm.at[pl.ds(e0, tail)], dst_t)
            pltpu.async_copy(hs_hbm.at[src_t], bufs[0].at[pl.ds(0, tail)],
                             semg[0]).wait()
            pltpu.sync_copy(bufs[0].at[pl.ds(0, tail)], acc.at[dst_t],
                            add=True)
        plsc.subcore_barrier()
        _rowcopy(sid, n,
                 lambda r, m: acc.at[pl.ds(r, m)],
                 lambda r, m: out_hbm.at[cid, pl.ds(r, m)])

    return agg_kernel


def _mk_deg(n, h, e):
    nw = _NC * _NS
    epw = e // nw
    nch = epw // _K
    tail = epw - nch * _K
    mesh = plsc.VectorSubcoreMesh(core_axis_name="c", subcore_axis_name="s")

    @functools.partial(
        pl.kernel,
        out_type=jax.ShapeDtypeStruct((_NC, n, h), jnp.float32),
        mesh=mesh,
        scratch_types=[
            pltpu.VMEM_SHARED((n, h), jnp.float32),
            pltpu.VMEM((nch, _K), jnp.int32),
            pltpu.VMEM((_K, h), jnp.float32),
            pltpu.VMEM((tail,), jnp.int32),
            pltpu.SemaphoreType.DMA,
            pltpu.SemaphoreType.DMA,
        ],
    )
    def deg_kernel(dst_hbm, ones_hbm, zeros_hbm, out_hbm,
                   acc, dsta, ones_v, dst_t, sems, semi):
        cid = lax.axis_index("c")
        sid = lax.axis_index("s")
        wid = cid * _NS + sid
        base = wid * epw

        _rowcopy(sid, n,
                 lambda r, m: zeros_hbm.at[pl.ds(r, m)],
                 lambda r, m: acc.at[pl.ds(r, m)])
        _stage_indices(dst_hbm, base, nch, dsta, semi,
                       extra=[(ones_hbm, ones_v)])
        plsc.subcore_barrier()

        def fire(g, c):
            pltpu.async_copy(ones_v, acc.at[dsta.at[g]], sems, add=True)
            return c

        lax.fori_loop(0, nch, fire, 0)

        def drain(g, c):
            pltpu.make_async_copy(ones_v, acc.at[dsta.at[g]], sems).wait()
            return c

        lax.fori_loop(0, nch, drain, 0)
        if tail:
            pltpu.sync_copy(dst_hbm.at[pl.ds(base + nch * _K, tail)], dst_t)
            pltpu.sync_copy(ones_v.at[pl.ds(0, tail)], acc.at[dst_t],
                            add=True)
        plsc.subcore_barrier()
        _rowcopy(sid, n,
                 lambda r, m: acc.at[pl.ds(r, m)],
                 lambda r, m: out_hbm.at[cid, pl.ds(r, m)])

    return deg_kernel


_BLK = 1000


def _pre(degs, x, w):
    n, d_in = x.shape
    hdn = w.shape[1]

    def body(degs_ref, x_ref, w_ref, dinv_ref, hs_ref):
        d = degs_ref[...]
        dinv = lax.rsqrt(d[0, :, 0:1] + d[1, :, 0:1] + 1.0)
        dinv_ref[...] = dinv
        hm = jnp.dot(x_ref[...], w_ref[...],
                     preferred_element_type=jnp.float32)
        hs_ref[...] = hm * dinv

    return pl.pallas_call(
        body,
        grid=(n // _BLK,),
        in_specs=[
            pl.BlockSpec((_NC, _BLK, d_in), lambda i: (0, i, 0)),
            pl.BlockSpec((_BLK, d_in), lambda i: (i, 0)),
            pl.BlockSpec((d_in, hdn), lambda i: (0, 0)),
        ],
        out_specs=[
            pl.BlockSpec((_BLK, 1), lambda i: (i, 0)),
            pl.BlockSpec((_BLK, hdn), lambda i: (i, 0)),
        ],
        out_shape=[
            jax.ShapeDtypeStruct((n, 1), jnp.float32),
            jax.ShapeDtypeStruct((n, hdn), jnp.float32),
        ],
    )(degs, x, w)


def _norm_block(accs, dinv, b, g, be):
    t = (accs[0] + accs[1]) * dinv + b
    t = jnp.maximum(t, 0.0)
    mu = jnp.mean(t, axis=-1, keepdims=True)
    tc = t - mu
    var = jnp.mean(tc * tc, axis=-1, keepdims=True)
    return tc * lax.rsqrt(var + 1e-5) * g + be


def _combine(accs, dinv, b, g, be, w):
    _, n, hdn = accs.shape
    hdn2 = w.shape[1]

    def body(accs_ref, dinv_ref, b_ref, g_ref, be_ref, w_ref, out_ref):
        dinv = dinv_ref[...]
        ln = _norm_block(accs_ref[...], dinv, b_ref[...], g_ref[...],
                         be_ref[...])
        out_ref[...] = jnp.dot(ln, w_ref[...],
                               preferred_element_type=jnp.float32) * dinv

    return pl.pallas_call(
        body,
        grid=(n // _BLK,),
        in_specs=[
            pl.BlockSpec((_NC, _BLK, hdn), lambda i: (0, i, 0)),
            pl.BlockSpec((_BLK, 1), lambda i: (i, 0)),
            pl.BlockSpec((1, hdn), lambda i: (0, 0)),
            pl.BlockSpec((1, hdn), lambda i: (0, 0)),
            pl.BlockSpec((1, hdn), lambda i: (0, 0)),
            pl.BlockSpec((hdn, hdn2), lambda i: (0, 0)),
        ],
        out_specs=pl.BlockSpec((_BLK, hdn2), lambda i: (i, 0)),
        out_shape=jax.ShapeDtypeStruct((n, hdn2), jnp.float32),
    )(accs, dinv, b, g, be, w)


def _final(accs, dinv, b, g, be, wc, bc):
    _, n, hdn = accs.shape
    od = wc.shape[1]

    def body(accs_ref, dinv_ref, b_ref, g_ref, be_ref, wc_ref, bc_ref,
             out_ref):
        ln = _norm_block(accs_ref[...], dinv_ref[...], b_ref[...], g_ref[...],
                         be_ref[...])
        out_ref[...] = jnp.dot(ln, wc_ref[...],
                               preferred_element_type=jnp.float32) + bc_ref[...]

    return pl.pallas_call(
        body,
        grid=(n // _BLK,),
        in_specs=[
            pl.BlockSpec((_NC, _BLK, hdn), lambda i: (0, i, 0)),
            pl.BlockSpec((_BLK, 1), lambda i: (i, 0)),
            pl.BlockSpec((1, hdn), lambda i: (0, 0)),
            pl.BlockSpec((1, hdn), lambda i: (0, 0)),
            pl.BlockSpec((1, hdn), lambda i: (0, 0)),
            pl.BlockSpec((hdn, od), lambda i: (0, 0)),
            pl.BlockSpec((1, od), lambda i: (0, 0)),
        ],
        out_specs=pl.BlockSpec((_BLK, od), lambda i: (i, 0)),
        out_shape=jax.ShapeDtypeStruct((n, od), jnp.float32),
    )(accs, dinv, b, g, be, wc, bc)


def kernel(x, edge_index, W1, b1, W2, b2, W3, b3, gamma, beta, Wc, bc):
    n, _ = x.shape
    e = edge_index.shape[1]
    hdn = W1.shape[1]
    src = edge_index[0]
    dst = edge_index[1]
    zeros2 = jnp.zeros((n, hdn), jnp.float32)
    onesk = jnp.ones((_K, hdn), jnp.float32)
    b1r, b2r, b3r = (v.reshape(1, -1) for v in (b1, b2, b3))
    gr, ber, bcr = gamma.reshape(1, -1), beta.reshape(1, -1), bc.reshape(1, -1)

    agg = _mk_agg(n, hdn, e)
    degs = _mk_deg(n, hdn, e)(dst, onesk, zeros2)
    dinv, hs = _pre(degs, x, W1)
    accs = agg(hs, src, dst, zeros2)
    hs = _combine(accs, dinv, b1r, gr, ber, W2)
    accs = agg(hs, src, dst, zeros2)
    hs = _combine(accs, dinv, b2r, gr, ber, W3)
    accs = agg(hs, src, dst, zeros2)
    return _final(accs, dinv, b3r, gr, ber, Wc, bcr)
